# Initial kernel scaffold; baseline (speedup 1.0000x reference)
import jax
import jax.numpy as jnp
from jax.experimental import pallas as pl


def kernel(batch_x, edge_index, glob_emb, W1, b1, g1, be1, W2, b2, W3, b3, g2, be2, W4, b4, Wl, bl):
    raise NotImplementedError("write your pallas kernel here")


# speedup vs baseline: 6.0159x; 6.0159x over previous
import functools

import jax
import jax.numpy as jnp
from jax import lax
from jax.experimental import pallas as pl
from jax.experimental.pallas import tpu as pltpu
from jax.experimental.pallas import tpu_sc as plsc

N = 10000
E = 320000
D = 128
H = 64

NC = 2
NS = 16
NW = NC * NS
BATCH = 128
TPT = -(-E // NW // BATCH) * BATCH
NB = TPT // BATCH
E_PAD = TPT * NW
RPT = -(-(N + 1) // (NS * 8)) * 8
NPAD = RPT * NS


def _sc_agg_body(y_hbm, src_hbm, dst_hbm, z_hbm, out_hbm,
                 src_v, dst_v, rows_v, acc, gsem):
    c = lax.axis_index("c")
    s = lax.axis_index("s")
    w = c * NS + s

    pltpu.sync_copy(src_hbm.at[w], src_v)
    pltpu.sync_copy(dst_hbm.at[w], dst_v)

    rstart = s * RPT
    pltpu.sync_copy(z_hbm, acc.at[pl.ds(rstart, RPT)])
    plsc.subcore_barrier()

    def body(j, carry):
        pltpu.async_copy(y_hbm.at[src_v.at[j]], rows_v, gsem).wait()
        pltpu.sync_copy(rows_v, acc.at[dst_v.at[j]], add=True)
        return carry

    lax.fori_loop(0, NB, body, 0)
    plsc.subcore_barrier()

    pltpu.sync_copy(acc.at[pl.ds(rstart, RPT)], out_hbm.at[c, pl.ds(rstart, RPT)])


@functools.partial(
    pl.kernel,
    out_type=jax.ShapeDtypeStruct((NC, NPAD, H), jnp.float32),
    mesh=plsc.VectorSubcoreMesh(core_axis_name="c", subcore_axis_name="s"),
    scratch_types=[
        pltpu.VMEM((NB, BATCH), jnp.int32),
        pltpu.VMEM((NB, BATCH), jnp.int32),
        pltpu.VMEM((BATCH, H), jnp.float32),
        pltpu.VMEM_SHARED((NPAD, H), jnp.float32),
        pltpu.SemaphoreType.DMA,
    ],
    compiler_params=pltpu.CompilerParams(use_tc_tiling_on_sc=False),
)
def _sc_agg(y_hbm, src_hbm, dst_hbm, z_hbm, out_hbm, src_v, dst_v, rows_v, acc, gsem):
    _sc_agg_body(y_hbm, src_hbm, dst_hbm, z_hbm, out_hbm,
                 src_v, dst_v, rows_v, acc, gsem)


def _mm_body(x_ref, w_ref, o_ref):
    o_ref[...] = jnp.dot(x_ref[...], w_ref[...], preferred_element_type=jnp.float32)


def _stage2_body(y_ref, p_ref, b1_ref, g1_ref, be1_ref, W2_ref, b2_ref, o_ref):
    t = y_ref[...] + p_ref[0, :N, :] + p_ref[1, :N, :] + b1_ref[...]
    mu = jnp.mean(t, axis=0, keepdims=True)
    var = jnp.mean((t - mu) ** 2, axis=0, keepdims=True)
    tn = (t - mu) * lax.rsqrt(var + 1e-5) * g1_ref[...] + be1_ref[...]
    r = jnp.maximum(tn, 0.0)
    x1 = jnp.dot(r, W2_ref[...], preferred_element_type=jnp.float32) + b2_ref[...]
    o_ref[...] = jnp.maximum(x1, 0.0)


def _stage3_body(x_ref, p_ref, b3_ref, g2_ref, be2_ref, W3_ref, W4_ref, b4_ref,
                 glob_ref, Wl_ref, bl_ref, o_ref):
    t = x_ref[...] + p_ref[0, :N, :] + p_ref[1, :N, :]
    u = jnp.dot(t, W3_ref[...], preferred_element_type=jnp.float32) + b3_ref[...]
    mu = jnp.mean(u, axis=0, keepdims=True)
    var = jnp.mean((u - mu) ** 2, axis=0, keepdims=True)
    un = (u - mu) * lax.rsqrt(var + 1e-5) * g2_ref[...] + be2_ref[...]
    r = jnp.maximum(un, 0.0)
    v = jnp.dot(r, W4_ref[...], preferred_element_type=jnp.float32) + b4_ref[...]
    o_ref[...] = (jnp.dot(v, Wl_ref[:H, :], preferred_element_type=jnp.float32)
                  + jnp.dot(glob_ref[...], Wl_ref[H:, :],
                            preferred_element_type=jnp.float32)
                  + bl_ref[...])


_mm = pl.pallas_call(_mm_body, out_shape=jax.ShapeDtypeStruct((N, H), jnp.float32))
_stage2 = pl.pallas_call(_stage2_body, out_shape=jax.ShapeDtypeStruct((N, H), jnp.float32))
_stage3 = pl.pallas_call(_stage3_body, out_shape=jax.ShapeDtypeStruct((N, D), jnp.float32))


def kernel(batch_x, edge_index, glob_emb, W1, b1, g1, be1, W2, b2, W3, b3,
           g2, be2, W4, b4, Wl, bl):
    src = edge_index[0]
    dst = edge_index[1]
    pad = E_PAD - E
    srcp = jnp.concatenate([src, jnp.zeros((pad,), jnp.int32)]).reshape(NW, NB, BATCH)
    dstp = jnp.concatenate([dst, jnp.full((pad,), N, jnp.int32)]).reshape(NW, NB, BATCH)
    zeros = jnp.zeros((RPT, H), jnp.float32)

    y0 = _mm(batch_x, W1)
    p0 = _sc_agg(y0, srcp, dstp, zeros)
    x1 = _stage2(y0, p0, b1.reshape(1, H), g1.reshape(1, H), be1.reshape(1, H),
                 W2, b2.reshape(1, H))
    p1 = _sc_agg(x1, srcp, dstp, zeros)
    out = _stage3(x1, p1, b3.reshape(1, H), g2.reshape(1, H), be2.reshape(1, H),
                  W3, W4, b4.reshape(1, H), glob_emb, Wl, bl.reshape(1, D))
    return out


# speedup vs baseline: 13.1020x; 2.1779x over previous
import functools

import jax
import jax.numpy as jnp
from jax import lax
from jax.experimental import pallas as pl
from jax.experimental.pallas import tpu as pltpu
from jax.experimental.pallas import tpu_sc as plsc

N = 10000
E = 320000
D = 128
H = 64

NC = 2
NS = 16
NW = NC * NS
BATCH = 128
NB = 2 * (-(-E // (NW * BATCH * 2)))
TPT = NB * BATCH
E_PAD = TPT * NW
RPT = -(-(N + 1) // (NS * 8)) * 8
NPAD = RPT * NS


def _sc_agg_body(y_hbm, src_hbm, dst_hbm, z_hbm, out_hbm,
                 src_v, dst_v, rows_a, rows_b, acc, sem_a, sem_b):
    c = lax.axis_index("c")
    s = lax.axis_index("s")
    w = c * NS + s

    pltpu.sync_copy(src_hbm.at[w], src_v)
    pltpu.sync_copy(dst_hbm.at[w], dst_v)

    rstart = s * RPT
    pltpu.sync_copy(z_hbm, acc.at[pl.ds(rstart, RPT)])
    plsc.subcore_barrier()

    pltpu.async_copy(y_hbm.at[src_v.at[0]], rows_a, sem_a)

    def body(i, carry):
        j0 = 2 * i
        j1 = j0 + 1
        pltpu.async_copy(y_hbm.at[src_v.at[j1]], rows_b, sem_b)
        pltpu.make_async_copy(y_hbm.at[src_v.at[j0]], rows_a, sem_a).wait()
        pltpu.sync_copy(rows_a, acc.at[dst_v.at[j0]], add=True)
        j2 = jnp.where(j1 + 1 < NB, j1 + 1, 0)
        pltpu.async_copy(y_hbm.at[src_v.at[j2]], rows_a, sem_a)
        pltpu.make_async_copy(y_hbm.at[src_v.at[j1]], rows_b, sem_b).wait()
        pltpu.sync_copy(rows_b, acc.at[dst_v.at[j1]], add=True)
        return carry

    lax.fori_loop(0, NB // 2, body, 0)
    pltpu.make_async_copy(y_hbm.at[src_v.at[0]], rows_a, sem_a).wait()
    plsc.subcore_barrier()

    pltpu.sync_copy(acc.at[pl.ds(rstart, RPT)], out_hbm.at[c, pl.ds(rstart, RPT)])


@functools.partial(
    pl.kernel,
    out_type=jax.ShapeDtypeStruct((NC, NPAD, H), jnp.float32),
    mesh=plsc.VectorSubcoreMesh(core_axis_name="c", subcore_axis_name="s"),
    scratch_types=[
        pltpu.VMEM((NB, BATCH), jnp.int32),
        pltpu.VMEM((NB, BATCH), jnp.int32),
        pltpu.VMEM((BATCH, H), jnp.float32),
        pltpu.VMEM((BATCH, H), jnp.float32),
        pltpu.VMEM_SHARED((NPAD, H), jnp.float32),
        pltpu.SemaphoreType.DMA,
        pltpu.SemaphoreType.DMA,
    ],
    compiler_params=pltpu.CompilerParams(use_tc_tiling_on_sc=False),
)
def _sc_agg(y_hbm, src_hbm, dst_hbm, z_hbm, out_hbm, src_v, dst_v,
            rows_a, rows_b, acc, sem_a, sem_b):
    _sc_agg_body(y_hbm, src_hbm, dst_hbm, z_hbm, out_hbm,
                 src_v, dst_v, rows_a, rows_b, acc, sem_a, sem_b)


def _mm_body(x_ref, w_ref, o_ref):
    o_ref[...] = jnp.dot(x_ref[...], w_ref[...], preferred_element_type=jnp.float32)


def _stage2_body(y_ref, p_ref, b1_ref, g1_ref, be1_ref, W2_ref, b2_ref, o_ref):
    t = y_ref[...] + p_ref[0, :N, :] + p_ref[1, :N, :] + b1_ref[...]
    mu = jnp.mean(t, axis=0, keepdims=True)
    var = jnp.mean((t - mu) ** 2, axis=0, keepdims=True)
    tn = (t - mu) * lax.rsqrt(var + 1e-5) * g1_ref[...] + be1_ref[...]
    r = jnp.maximum(tn, 0.0)
    x1 = jnp.dot(r, W2_ref[...], preferred_element_type=jnp.float32) + b2_ref[...]
    o_ref[...] = jnp.maximum(x1, 0.0)


def _stage3_body(x_ref, p_ref, b3_ref, g2_ref, be2_ref, W3_ref, W4_ref, b4_ref,
                 glob_ref, Wl_ref, bl_ref, o_ref):
    t = x_ref[...] + p_ref[0, :N, :] + p_ref[1, :N, :]
    u = jnp.dot(t, W3_ref[...], preferred_element_type=jnp.float32) + b3_ref[...]
    mu = jnp.mean(u, axis=0, keepdims=True)
    var = jnp.mean((u - mu) ** 2, axis=0, keepdims=True)
    un = (u - mu) * lax.rsqrt(var + 1e-5) * g2_ref[...] + be2_ref[...]
    r = jnp.maximum(un, 0.0)
    v = jnp.dot(r, W4_ref[...], preferred_element_type=jnp.float32) + b4_ref[...]
    o_ref[...] = (jnp.dot(v, Wl_ref[:H, :], preferred_element_type=jnp.float32)
                  + jnp.dot(glob_ref[...], Wl_ref[H:, :],
                            preferred_element_type=jnp.float32)
                  + bl_ref[...])


_mm = pl.pallas_call(_mm_body, out_shape=jax.ShapeDtypeStruct((N, H), jnp.float32))
_stage2 = pl.pallas_call(_stage2_body, out_shape=jax.ShapeDtypeStruct((N, H), jnp.float32))
_stage3 = pl.pallas_call(_stage3_body, out_shape=jax.ShapeDtypeStruct((N, D), jnp.float32))


def kernel(batch_x, edge_index, glob_emb, W1, b1, g1, be1, W2, b2, W3, b3,
           g2, be2, W4, b4, Wl, bl):
    src = edge_index[0]
    dst = edge_index[1]
    pad = E_PAD - E
    ar = jnp.arange(pad, dtype=jnp.int32)
    srcp = jnp.concatenate([src, ar % N]).reshape(NW, NB, BATCH)
    dstp = jnp.concatenate([dst, N + ar % (NPAD - N)]).reshape(NW, NB, BATCH)
    zeros = jnp.zeros((RPT, H), jnp.float32)

    y0 = _mm(batch_x, W1)
    p0 = _sc_agg(y0, srcp, dstp, zeros)
    x1 = _stage2(y0, p0, b1.reshape(1, H), g1.reshape(1, H), be1.reshape(1, H),
                 W2, b2.reshape(1, H))
    p1 = _sc_agg(x1, srcp, dstp, zeros)
    out = _stage3(x1, p1, b3.reshape(1, H), g2.reshape(1, H), be2.reshape(1, H),
                  W3, W4, b4.reshape(1, H), glob_emb, Wl, bl.reshape(1, D))
    return out


# speedup vs baseline: 14.4324x; 1.1015x over previous
import functools

import jax
import jax.numpy as jnp
from jax import lax
from jax.experimental import pallas as pl
from jax.experimental.pallas import tpu as pltpu
from jax.experimental.pallas import tpu_sc as plsc

N = 10000
E = 320000
D = 128
H = 64

NC = 2
NS = 16
NW = NC * NS
BATCH = 128
G = 4
NB = 2 * G * (-(-E // (NW * BATCH * 2 * G)))
TPT = NB * BATCH
E_PAD = TPT * NW
RPT = -(-(N + 1) // (NS * 8)) * 8
NPAD = RPT * NS


def _sc_agg_body(y_hbm, src_hbm, dst_hbm, z_hbm, out_hbm,
                 src_v, dst_v, rows_a, rows_b, acc, sem_a, sem_b,
                 ssem_a, ssem_b):
    c = lax.axis_index("c")
    s = lax.axis_index("s")
    w = c * NS + s

    pltpu.sync_copy(src_hbm.at[w], src_v)
    pltpu.sync_copy(dst_hbm.at[w], dst_v)

    rstart = s * RPT
    pltpu.sync_copy(z_hbm, acc.at[pl.ds(rstart, RPT)])
    plsc.subcore_barrier()

    def gathers(g0, buf, sem):
        for k in range(G):
            pltpu.async_copy(y_hbm.at[src_v.at[g0 + k]],
                             buf.at[pl.ds(k * BATCH, BATCH)], sem)

    def wait_gathers(g0, buf, sem):
        for k in range(G):
            pltpu.make_async_copy(y_hbm.at[src_v.at[g0 + k]],
                                  buf.at[pl.ds(k * BATCH, BATCH)], sem).wait()

    def scatters(g0, buf, sem):
        for k in range(G):
            pltpu.async_copy(buf.at[pl.ds(k * BATCH, BATCH)],
                             acc.at[dst_v.at[g0 + k]], sem, add=True)
        for k in range(G):
            pltpu.make_async_copy(buf.at[pl.ds(k * BATCH, BATCH)],
                                  acc.at[dst_v.at[g0 + k]], sem).wait()

    gathers(0, rows_a, sem_a)

    def body(i, carry):
        g = 2 * G * i
        gathers(g + G, rows_b, sem_b)
        wait_gathers(g, rows_a, sem_a)
        scatters(g, rows_a, ssem_a)
        g2 = jnp.where(g + 2 * G < NB, g + 2 * G, 0)
        gathers(g2, rows_a, sem_a)
        wait_gathers(g + G, rows_b, sem_b)
        scatters(g + G, rows_b, ssem_b)
        return carry

    lax.fori_loop(0, NB // (2 * G), body, 0)
    wait_gathers(0, rows_a, sem_a)
    plsc.subcore_barrier()

    pltpu.sync_copy(acc.at[pl.ds(rstart, RPT)], out_hbm.at[c, pl.ds(rstart, RPT)])


@functools.partial(
    pl.kernel,
    out_type=jax.ShapeDtypeStruct((NC, NPAD, H), jnp.float32),
    mesh=plsc.VectorSubcoreMesh(core_axis_name="c", subcore_axis_name="s"),
    scratch_types=[
        pltpu.VMEM((NB, BATCH), jnp.int32),
        pltpu.VMEM((NB, BATCH), jnp.int32),
        pltpu.VMEM((G * BATCH, H), jnp.float32),
        pltpu.VMEM((G * BATCH, H), jnp.float32),
        pltpu.VMEM_SHARED((NPAD, H), jnp.float32),
        pltpu.SemaphoreType.DMA,
        pltpu.SemaphoreType.DMA,
        pltpu.SemaphoreType.DMA,
        pltpu.SemaphoreType.DMA,
    ],
    compiler_params=pltpu.CompilerParams(use_tc_tiling_on_sc=False),
)
def _sc_agg(y_hbm, src_hbm, dst_hbm, z_hbm, out_hbm, src_v, dst_v,
            rows_a, rows_b, acc, sem_a, sem_b, ssem_a, ssem_b):
    _sc_agg_body(y_hbm, src_hbm, dst_hbm, z_hbm, out_hbm,
                 src_v, dst_v, rows_a, rows_b, acc, sem_a, sem_b, ssem_a, ssem_b)


def _mm_body(x_ref, w_ref, o_ref):
    o_ref[...] = jnp.dot(x_ref[...], w_ref[...], preferred_element_type=jnp.float32)


def _stage2_body(y_ref, p_ref, b1_ref, g1_ref, be1_ref, W2_ref, b2_ref, o_ref):
    t = y_ref[...] + p_ref[0, :N, :] + p_ref[1, :N, :] + b1_ref[...]
    mu = jnp.mean(t, axis=0, keepdims=True)
    var = jnp.mean(t * t, axis=0, keepdims=True) - mu * mu
    tn = (t - mu) * lax.rsqrt(var + 1e-5) * g1_ref[...] + be1_ref[...]
    r = jnp.maximum(tn, 0.0)
    x1 = jnp.dot(r, W2_ref[...], preferred_element_type=jnp.float32) + b2_ref[...]
    o_ref[...] = jnp.maximum(x1, 0.0)


def _stage3_body(x_ref, p_ref, b3_ref, g2_ref, be2_ref, W3_ref, W4_ref, b4_ref,
                 glob_ref, Wl_ref, bl_ref, o_ref):
    t = x_ref[...] + p_ref[0, :N, :] + p_ref[1, :N, :]
    u = jnp.dot(t, W3_ref[...], preferred_element_type=jnp.float32) + b3_ref[...]
    mu = jnp.mean(u, axis=0, keepdims=True)
    var = jnp.mean(u * u, axis=0, keepdims=True) - mu * mu
    un = (u - mu) * lax.rsqrt(var + 1e-5) * g2_ref[...] + be2_ref[...]
    r = jnp.maximum(un, 0.0)
    v = jnp.dot(r, W4_ref[...], preferred_element_type=jnp.float32) + b4_ref[...]
    o_ref[...] = (jnp.dot(v, Wl_ref[:H, :], preferred_element_type=jnp.float32)
                  + jnp.dot(glob_ref[...], Wl_ref[H:, :],
                            preferred_element_type=jnp.float32)
                  + bl_ref[...])


_mm = pl.pallas_call(_mm_body, out_shape=jax.ShapeDtypeStruct((N, H), jnp.float32))
_stage2 = pl.pallas_call(_stage2_body, out_shape=jax.ShapeDtypeStruct((N, H), jnp.float32))
_stage3 = pl.pallas_call(_stage3_body, out_shape=jax.ShapeDtypeStruct((N, D), jnp.float32))


def kernel(batch_x, edge_index, glob_emb, W1, b1, g1, be1, W2, b2, W3, b3,
           g2, be2, W4, b4, Wl, bl):
    src = edge_index[0]
    dst = edge_index[1]
    pad = E_PAD - E
    ar = jnp.arange(pad, dtype=jnp.int32)
    srcp = jnp.concatenate([src, ar % N]).reshape(NW, NB, BATCH)
    dstp = jnp.concatenate([dst, N + ar % (NPAD - N)]).reshape(NW, NB, BATCH)
    zeros = jnp.zeros((RPT, H), jnp.float32)

    y0 = _mm(batch_x, W1)
    p0 = _sc_agg(y0, srcp, dstp, zeros)
    x1 = _stage2(y0, p0, b1.reshape(1, H), g1.reshape(1, H), be1.reshape(1, H),
                 W2, b2.reshape(1, H))
    p1 = _sc_agg(x1, srcp, dstp, zeros)
    out = _stage3(x1, p1, b3.reshape(1, H), g2.reshape(1, H), be2.reshape(1, H),
                  W3, W4, b4.reshape(1, H), glob_emb, Wl, bl.reshape(1, D))
    return out


# speedup vs baseline: 17.2915x; 1.1981x over previous
import functools

import jax
import jax.numpy as jnp
from jax import lax
from jax.experimental import pallas as pl
from jax.experimental.pallas import tpu as pltpu
from jax.experimental.pallas import tpu_sc as plsc

N = 10000
E = 320000
D = 128
H = 64
P = N // 2

NC = 2
NS = 16
NW = NC * NS
BATCH = 128
G = 3
TB = E // BATCH
NB = (TB // NW // (2 * G)) * (2 * G)
XTRA = TB - NB * NW
RPT = -(-N // (NS * 8)) * 8
NPAD = RPT * NS


def _sc_agg_body(y_hbm, e_hbm, z_hbm, out_hbm,
                 src_v, dst_v, rows_a, rows_b, acc, sem_a, sem_b,
                 ssem_a, ssem_b):
    c = lax.axis_index("c")
    s = lax.axis_index("s")
    w = c * NS + s
    bstart = NB * w + jnp.minimum(w, XTRA)

    pltpu.sync_copy(e_hbm.at[0, pl.ds(bstart, NB)], src_v.at[pl.ds(0, NB)])
    pltpu.sync_copy(e_hbm.at[1, pl.ds(bstart, NB)], dst_v.at[pl.ds(0, NB)])

    @pl.when(w < XTRA)
    def _():
        pltpu.sync_copy(e_hbm.at[0, pl.ds(bstart + NB, 1)], src_v.at[pl.ds(NB, 1)])
        pltpu.sync_copy(e_hbm.at[1, pl.ds(bstart + NB, 1)], dst_v.at[pl.ds(NB, 1)])

    rstart = s * RPT
    pltpu.sync_copy(z_hbm, acc.at[pl.ds(rstart, RPT)])
    plsc.subcore_barrier()

    def gathers(g0, buf, sem):
        for k in range(G):
            pltpu.async_copy(y_hbm.at[src_v.at[g0 + k]],
                             buf.at[pl.ds(k * BATCH, BATCH)], sem)

    def wait_gathers(g0, buf, sem):
        for k in range(G):
            pltpu.make_async_copy(y_hbm.at[src_v.at[g0 + k]],
                                  buf.at[pl.ds(k * BATCH, BATCH)], sem).wait()

    def scatters(g0, buf, sem):
        for k in range(G):
            pltpu.async_copy(buf.at[pl.ds(k * BATCH, BATCH)],
                             acc.at[dst_v.at[g0 + k]], sem, add=True)
        for k in range(G):
            pltpu.make_async_copy(buf.at[pl.ds(k * BATCH, BATCH)],
                                  acc.at[dst_v.at[g0 + k]], sem).wait()

    gathers(0, rows_a, sem_a)

    def body(i, carry):
        g = 2 * G * i
        gathers(g + G, rows_b, sem_b)
        wait_gathers(g, rows_a, sem_a)
        scatters(g, rows_a, ssem_a)
        g2 = jnp.where(g + 2 * G < NB, g + 2 * G, 0)
        gathers(g2, rows_a, sem_a)
        wait_gathers(g + G, rows_b, sem_b)
        scatters(g + G, rows_b, ssem_b)
        return carry

    lax.fori_loop(0, NB // (2 * G), body, 0)
    wait_gathers(0, rows_a, sem_a)

    @pl.when(w < XTRA)
    def _():
        pltpu.async_copy(y_hbm.at[src_v.at[NB]],
                         rows_a.at[pl.ds(0, BATCH)], sem_a)
        pltpu.make_async_copy(y_hbm.at[src_v.at[NB]],
                              rows_a.at[pl.ds(0, BATCH)], sem_a).wait()
        pltpu.sync_copy(rows_a.at[pl.ds(0, BATCH)],
                        acc.at[dst_v.at[NB]], add=True)

    plsc.subcore_barrier()
    pltpu.sync_copy(acc.at[pl.ds(rstart, RPT)], out_hbm.at[c, pl.ds(rstart, RPT)])


@functools.partial(
    pl.kernel,
    out_type=jax.ShapeDtypeStruct((NC, NPAD, H), jnp.float32),
    mesh=plsc.VectorSubcoreMesh(core_axis_name="c", subcore_axis_name="s"),
    scratch_types=[
        pltpu.VMEM((NB + 1, BATCH), jnp.int32),
        pltpu.VMEM((NB + 1, BATCH), jnp.int32),
        pltpu.VMEM((G * BATCH, H), jnp.float32),
        pltpu.VMEM((G * BATCH, H), jnp.float32),
        pltpu.VMEM_SHARED((NPAD, H), jnp.float32),
        pltpu.SemaphoreType.DMA,
        pltpu.SemaphoreType.DMA,
        pltpu.SemaphoreType.DMA,
        pltpu.SemaphoreType.DMA,
    ],
    compiler_params=pltpu.CompilerParams(use_tc_tiling_on_sc=False),
)
def _sc_agg(y_hbm, e_hbm, z_hbm, out_hbm, src_v, dst_v,
            rows_a, rows_b, acc, sem_a, sem_b, ssem_a, ssem_b):
    _sc_agg_body(y_hbm, e_hbm, z_hbm, out_hbm,
                 src_v, dst_v, rows_a, rows_b, acc, sem_a, sem_b, ssem_a, ssem_b)


def _fold_bn(t, g2w, be2w):
    colsum = jnp.sum(t, axis=0, keepdims=True)
    colsq = jnp.sum(t * t, axis=0, keepdims=True)
    mu = (colsum[:, :H] + colsum[:, H:]) / N
    var = (colsq[:, :H] + colsq[:, H:]) / N - mu * mu
    scale = lax.rsqrt(var + 1e-5)
    mu2 = jnp.concatenate([mu, mu], axis=1)
    scale2 = jnp.concatenate([scale, scale], axis=1)
    return (t - mu2) * scale2 * g2w + be2w


def _mm_body(x_ref, w_ref, o_ref):
    o_ref[...] = jnp.dot(x_ref[...], w_ref[...], preferred_element_type=jnp.float32)


def _stage2_body(y_ref, p_ref, b1_ref, g1_ref, be1_ref, W2b_ref, b2_ref, o_ref):
    t = y_ref[...] + p_ref[0, :P, :] + p_ref[1, :P, :] + b1_ref[...]
    r = jnp.maximum(_fold_bn(t, g1_ref[...], be1_ref[...]), 0.0)
    x1 = jnp.dot(r, W2b_ref[...], preferred_element_type=jnp.float32) + b2_ref[...]
    o_ref[...] = jnp.maximum(x1, 0.0)


def _stage3_body(x_ref, p_ref, b3_ref, g2_ref, be2_ref, W3b_ref, W4b_ref, b4_ref,
                 globp_ref, Wltb_ref, Wlbb_ref, bl_ref, o_ref):
    t = x_ref[...] + p_ref[0, :P, :] + p_ref[1, :P, :]
    u = jnp.dot(t, W3b_ref[...], preferred_element_type=jnp.float32) + b3_ref[...]
    r = jnp.maximum(_fold_bn(u, g2_ref[...], be2_ref[...]), 0.0)
    v = jnp.dot(r, W4b_ref[...], preferred_element_type=jnp.float32) + b4_ref[...]
    o_ref[...] = (jnp.dot(v, Wltb_ref[...], preferred_element_type=jnp.float32)
                  + jnp.dot(globp_ref[...], Wlbb_ref[...],
                            preferred_element_type=jnp.float32)
                  + bl_ref[...])


_mm = pl.pallas_call(_mm_body, out_shape=jax.ShapeDtypeStruct((N, H), jnp.float32))
_stage2 = pl.pallas_call(_stage2_body, out_shape=jax.ShapeDtypeStruct((P, 2 * H), jnp.float32))
_stage3 = pl.pallas_call(_stage3_body, out_shape=jax.ShapeDtypeStruct((P, 2 * D), jnp.float32))


def _dup(v):
    return jnp.concatenate([v, v]).reshape(1, -1)


def _blockdiag(w):
    z = jnp.zeros(w.shape, jnp.float32)
    return jnp.concatenate([jnp.concatenate([w, z], axis=1),
                            jnp.concatenate([z, w], axis=1)], axis=0)


def kernel(batch_x, edge_index, glob_emb, W1, b1, g1, be1, W2, b2, W3, b3,
           g2, be2, W4, b4, Wl, bl):
    e3 = edge_index.reshape(2, TB, BATCH)
    zeros = jnp.zeros((RPT, H), jnp.float32)

    y0p = _mm(batch_x, W1).reshape(P, 2 * H)
    p0 = _sc_agg(y0p.reshape(N, H), e3, zeros)
    x1p = _stage2(y0p, p0.reshape(NC, NPAD // 2, 2 * H), _dup(b1), _dup(g1),
                  _dup(be1), _blockdiag(W2), _dup(b2))
    p1 = _sc_agg(x1p.reshape(N, H), e3, zeros)
    outp = _stage3(x1p, p1.reshape(NC, NPAD // 2, 2 * H), _dup(b3), _dup(g2),
                   _dup(be2), _blockdiag(W3), _blockdiag(W4), _dup(b4),
                   glob_emb.reshape(P, 2 * H), _blockdiag(Wl[:H, :]),
                   _blockdiag(Wl[H:, :]), _dup(bl))
    return outp.reshape(N, D)
